# baseline (speedup 1.0000x reference)
import jax
import jax.numpy as jnp
from jax import lax
from jax.experimental import pallas as pl
from jax.experimental.pallas import tpu as pltpu
from jax.experimental.pallas import tpu_sc as plsc

_NC = 2
_NS = 16
_NW = _NC * _NS
_CH = 32768


def _sc_body(a_hbm, idx_hbm, out_hbm, idxbuf, buf0, buf1, s_r0, s_r1, s_w0, s_w1):
    n = a_hbm.shape[0]
    span = n // _NW
    nch = span // _CH
    cid = lax.axis_index("c")
    sid = lax.axis_index("s")
    wid = sid * _NC + cid
    base = wid * span

    pltpu.sync_copy(idx_hbm, idxbuf.at[pl.ds(0, 1)])
    lanes = lax.broadcasted_iota(jnp.int32, (16,), 0)
    v = idxbuf[...]
    idx = jnp.max(jnp.where(lanes == 0, v, jnp.int32(-2147483647)))

    bufs = (buf0, buf1)
    rsem = (s_r0, s_r1)
    wsem = (s_w0, s_w1)

    def rd(i, b):
        return pltpu.make_async_copy(
            a_hbm.at[pl.ds(base + i * _CH, _CH)], bufs[b], rsem[b])

    def wr(i, b):
        return pltpu.make_async_copy(
            bufs[b], out_hbm.at[pl.ds(base + i * _CH, _CH)], wsem[b])

    rd(0, 0).start()
    for i in range(nch):
        b = i % 2
        rd(i, b).wait()
        if i + 1 < nch:
            if i >= 1:
                wr(i - 1, 1 - b).wait()
            rd(i + 1, 1 - b).start()

        cstart = base + i * _CH
        for t in range(2):
            rel = idx + t - cstart

            @pl.when(jnp.logical_and(rel >= 0, rel < _CH))
            def _fix(rel=rel, b=b):
                w0 = (rel // 16) * 16
                lane = rel - w0
                win = bufs[b][pl.ds(w0, 16)]
                bufs[b][pl.ds(w0, 16)] = jnp.where(
                    lanes == lane, jnp.float32(0), win)

        wr(i, b).start()
    wr(nch - 1, (nch - 1) % 2).wait()
    if nch >= 2:
        wr(nch - 2, nch % 2).wait()


def kernel(a, idx):
    n = a.shape[0]
    idx32 = idx.astype(jnp.int32)
    mesh = plsc.VectorSubcoreMesh(core_axis_name="c", subcore_axis_name="s")
    f = pl.kernel(
        _sc_body,
        out_type=jax.ShapeDtypeStruct((n,), a.dtype),
        mesh=mesh,
        compiler_params=pltpu.CompilerParams(needs_layout_passes=False),
        scratch_types=[
            pltpu.VMEM((16,), jnp.int32),
            pltpu.VMEM((_CH,), jnp.float32),
            pltpu.VMEM((_CH,), jnp.float32),
            pltpu.SemaphoreType.DMA,
            pltpu.SemaphoreType.DMA,
            pltpu.SemaphoreType.DMA,
            pltpu.SemaphoreType.DMA,
        ],
    )
    return f(a, idx32)
